# baseline (speedup 1.0000x reference)
import functools

import jax
import jax.numpy as jnp
from jax import lax
from jax.experimental import pallas as pl
from jax.experimental.pallas import tpu as pltpu
from jax.experimental.pallas import tpu_sc as plsc

N = 10000
E = 320000
F_IN = 128
H = 16
C = 40

NPAD = 10240
NC = 2
NS = 16
NW = NC * NS
CHUNK = 128
NCH_TOTAL = E // CHUNK
NCH_BASE = NCH_TOTAL // NW
NCH_EXTRA = NCH_TOTAL % NW
RPT = NPAD // NS
DEPTH = 32
GA = 24
SD = 8

RPACK = 128 // H
RB = 1024
PKR = RB // RPACK
_GRID = NPAD // RB
_PK = NPAD // RPACK


def _agg_body(with_deg, *refs):
    if with_deg:
        (ei_hbm, tbl_hbm, zacc_hbm, zdeg_hbm, ones_hbm,
         acc_out, deg_out,
         srcv, dstv, rows, onesv, degv, degrep,
         acc_sh, deg_sh, sem_g, sem_s, sem_d) = refs
    else:
        (ei_hbm, tbl_hbm, zacc_hbm,
         acc_out,
         srcv, dstv, rows, acc_sh, sem_g, sem_s) = refs

    c = lax.axis_index("c")
    s = lax.axis_index("s")
    w = c * NS + s
    row0 = NCH_BASE * w + jnp.minimum(w, NCH_EXTRA)
    nch = NCH_BASE + jnp.where(w < NCH_EXTRA, 1, 0)

    zb = s * RPT
    init_cps = [
        pltpu.make_async_copy(zacc_hbm, acc_sh.at[pl.ds(zb, RPT)], sem_g),
        pltpu.make_async_copy(ei_hbm.at[0, pl.ds(row0, NCH_BASE)],
                              srcv.at[pl.ds(0, NCH_BASE)], sem_g),
        pltpu.make_async_copy(ei_hbm.at[1, pl.ds(row0, NCH_BASE)],
                              dstv.at[pl.ds(0, NCH_BASE)], sem_g),
    ]
    if with_deg:
        init_cps += [
            pltpu.make_async_copy(zdeg_hbm, deg_sh.at[pl.ds(zb, RPT)], sem_g),
            pltpu.make_async_copy(ones_hbm, onesv, sem_g),
        ]
    for cp in init_cps:
        cp.start()

    @pl.when(w < NCH_EXTRA)
    def _():
        pltpu.sync_copy(ei_hbm.at[0, row0 + NCH_BASE], srcv.at[NCH_BASE])
        pltpu.sync_copy(ei_hbm.at[1, row0 + NCH_BASE], dstv.at[NCH_BASE])

    for cp in init_cps:
        cp.wait()

    plsc.subcore_barrier()

    def g_slot(i):
        return rows.at[pl.ds(lax.rem(i, DEPTH) * CHUNK, CHUNK)]

    for k in range(GA):
        @pl.when(k < nch)
        def _(k=k):
            pltpu.make_async_copy(tbl_hbm.at[srcv.at[k]],
                                  rows.at[pl.ds(k * CHUNK, CHUNK)],
                                  sem_g).start()

    def chunk_step(i):
        @pl.when(i >= SD)
        def _():
            pltpu.make_async_copy(g_slot(i - SD),
                                  acc_sh.at[dstv.at[i - SD]], sem_s).wait()
            if with_deg:
                pltpu.make_async_copy(onesv, deg_sh.at[dstv.at[i - SD]],
                                      sem_d).wait()

        @pl.when(i + GA < nch)
        def _():
            pltpu.make_async_copy(tbl_hbm.at[srcv.at[i + GA]],
                                  g_slot(i + GA), sem_g).start()

        pltpu.make_async_copy(tbl_hbm.at[srcv.at[i]], g_slot(i), sem_g).wait()
        pltpu.async_copy(g_slot(i), acc_sh.at[dstv.at[i]], sem_s, add=True)
        if with_deg:
            pltpu.async_copy(onesv, deg_sh.at[dstv.at[i]], sem_d, add=True)

    def step(i, carry):
        chunk_step(i)
        return carry

    lax.fori_loop(0, nch, step, 0)

    def tail(i, carry):
        @pl.when(i >= 0)
        def _():
            pltpu.make_async_copy(g_slot(i), acc_sh.at[dstv.at[i]],
                                  sem_s).wait()
            if with_deg:
                pltpu.make_async_copy(onesv, deg_sh.at[dstv.at[i]],
                                      sem_d).wait()
        return carry

    lax.fori_loop(jnp.maximum(nch - SD, 0), nch, tail, 0)

    plsc.subcore_barrier()
    pltpu.sync_copy(acc_sh.at[pl.ds(zb, RPT)], acc_out.at[c, pl.ds(zb, RPT)])
    if with_deg:
        pltpu.sync_copy(deg_sh.at[pl.ds(zb, RPT)], degv)

        def rep(i, carry):
            v = degv[pl.ds(i * H, H)]
            for k in range(H):
                degrep[i * H + k, :] = jnp.full((H,), v[k], jnp.float32)
            return carry

        lax.fori_loop(0, RPT // H, rep, 0)
        pltpu.sync_copy(degrep, deg_out.at[c, pl.ds(zb, RPT)])


def _make_agg(with_deg):
    mesh = plsc.VectorSubcoreMesh(
        core_axis_name="c", subcore_axis_name="s",
        num_cores=NC, num_subcores=NS)
    out_type = [jax.ShapeDtypeStruct((NC, NPAD, H), jnp.float32)]
    scratch = [
        pltpu.VMEM((NCH_BASE + 1, CHUNK), jnp.int32),
        pltpu.VMEM((NCH_BASE + 1, CHUNK), jnp.int32),
        pltpu.VMEM((DEPTH * CHUNK, H), jnp.float32),
    ]
    if with_deg:
        out_type.append(jax.ShapeDtypeStruct((NC, NPAD, H), jnp.float32))
        scratch += [
            pltpu.VMEM((CHUNK,), jnp.float32),
            pltpu.VMEM((RPT,), jnp.float32),
            pltpu.VMEM((RPT, H), jnp.float32),
        ]
    scratch.append(pltpu.VMEM_SHARED((NPAD, H), jnp.float32))
    if with_deg:
        scratch.append(pltpu.VMEM_SHARED((NPAD,), jnp.float32))
    scratch += [pltpu.SemaphoreType.DMA, pltpu.SemaphoreType.DMA]
    if with_deg:
        scratch.append(pltpu.SemaphoreType.DMA)
    return pl.kernel(
        functools.partial(_agg_body, with_deg),
        out_type=out_type,
        mesh=mesh,
        scratch_types=scratch,
        compiler_params=pltpu.CompilerParams(use_tc_tiling_on_sc=False),
    )


def _mm1_body(x_ref, w_ref, b_ref, ya_ref, xr_ref):
    z = jnp.dot(x_ref[...], w_ref[...], preferred_element_type=jnp.float32)
    nl = lax.broadcasted_iota(jnp.int32, (RB, 128), 0)
    li = lax.broadcasted_iota(jnp.int32, (RB, 128), 1)
    msk = jnp.where((li >> 4) == (nl & 7), 1.0, 0.0)
    msk2 = jnp.concatenate([msk, msk], axis=1)
    qi = lax.broadcasted_iota(jnp.int32, (PKR, RB), 0)
    ni = lax.broadcasted_iota(jnp.int32, (PKR, RB), 1)
    a8t = jnp.where((ni >> 3) == qi, 1.0, 0.0)
    yz = jnp.dot(a8t, z * msk2, preferred_element_type=jnp.float32)
    ya_ref[...] = yz[:, :128]
    xr_ref[...] = yz[:, 128:] + b_ref[...]


def _fuse1_body(acc_ref, deg_ref, xr_ref, h_ref):
    a = acc_ref[...]
    d = deg_ref[...]
    rinv = 1.0 / jnp.maximum(d[0] + d[1], 1.0)
    h_ref[...] = jnp.maximum((a[0] + a[1]) * rinv + xr_ref[...], 0.0)


def _fuse2_body(acc_ref, deg_ref, h_ref, w_ref, b_ref, out_ref):
    a = acc_ref[...]
    d = deg_ref[...]
    rinv = 1.0 / jnp.maximum(d[0] + d[1], 1.0)
    mean2 = (a[0] + a[1]) * rinv
    m2t = mean2.T
    ht = h_ref[...].T
    qi = lax.broadcasted_iota(jnp.int32, (PKR, RB), 0)
    ni = lax.broadcasted_iota(jnp.int32, (PKR, RB), 1)
    a8x = jnp.where(qi == (ni >> 3), 1.0, 0.0)
    li = lax.broadcasted_iota(jnp.int32, (128, RB), 0)
    nl = lax.broadcasted_iota(jnp.int32, (128, RB), 1)
    mskx = jnp.where((li >> 4) == (nl & 7), 1.0, 0.0)
    m2x = jnp.dot(m2t, a8x, preferred_element_type=jnp.float32) * mskx
    hx = jnp.dot(ht, a8x, preferred_element_type=jnp.float32) * mskx
    hw = jnp.concatenate([m2x, hx], axis=0)
    o = jnp.dot(w_ref[...], hw, preferred_element_type=jnp.float32) + b_ref[...]
    m = jnp.max(o, axis=0, keepdims=True)
    e = jnp.exp(o - m)
    lse = jnp.log(jnp.sum(e, axis=0, keepdims=True))
    out_ref[...] = (o - m) - lse


def _mm1(x, wrep, b1x):
    return pl.pallas_call(
        _mm1_body,
        grid=(_GRID,),
        in_specs=[
            pl.BlockSpec((RB, F_IN), lambda i: (i, 0)),
            pl.BlockSpec((F_IN, 256), lambda i: (0, 0)),
            pl.BlockSpec((1, 128), lambda i: (0, 0)),
        ],
        out_specs=[
            pl.BlockSpec((PKR, 128), lambda i: (i, 0)),
            pl.BlockSpec((PKR, 128), lambda i: (i, 0)),
        ],
        out_shape=[
            jax.ShapeDtypeStruct((_PK, 128), jnp.float32),
            jax.ShapeDtypeStruct((_PK, 128), jnp.float32),
        ],
    )(x, wrep, b1x)


def _fuse1(accp, degp, xrp):
    return pl.pallas_call(
        _fuse1_body,
        grid=(_GRID,),
        in_specs=[
            pl.BlockSpec((NC, PKR, 128), lambda i: (0, i, 0)),
            pl.BlockSpec((NC, PKR, 128), lambda i: (0, i, 0)),
            pl.BlockSpec((PKR, 128), lambda i: (i, 0)),
        ],
        out_specs=pl.BlockSpec((PKR, 128), lambda i: (i, 0)),
        out_shape=jax.ShapeDtypeStruct((_PK, 128), jnp.float32),
    )(accp, degp, xrp)


def _fuse2(accp2, degp, hp, w2x, b2r):
    return pl.pallas_call(
        _fuse2_body,
        grid=(_GRID,),
        in_specs=[
            pl.BlockSpec((NC, PKR, 128), lambda i: (0, i, 0)),
            pl.BlockSpec((NC, PKR, 128), lambda i: (0, i, 0)),
            pl.BlockSpec((PKR, 128), lambda i: (i, 0)),
            pl.BlockSpec((C, 256), lambda i: (0, 0)),
            pl.BlockSpec((C, 1), lambda i: (0, 0)),
        ],
        out_specs=pl.BlockSpec((C, RB), lambda i: (0, i)),
        out_shape=jax.ShapeDtypeStruct((C, N), jnp.float32),
    )(accp2, degp, hp, w2x, b2r)


_agg_with_deg = _make_agg(True)
_agg_no_deg = _make_agg(False)


@jax.jit
def kernel(x, edge_index, W1_l, W1_r, b1, W2_l, W2_r, b2):
    f32 = jnp.float32
    ei3 = edge_index.reshape(2, NCH_TOTAL, CHUNK)

    wrep = jnp.concatenate(
        [jnp.tile(W1_l, (1, RPACK)), jnp.tile(W1_r, (1, RPACK))],
        axis=1)
    b1x = jnp.tile(b1, RPACK).reshape(1, 128)
    w2x = jnp.concatenate(
        [jnp.tile(W2_l, (RPACK, 1)), jnp.tile(W2_r, (RPACK, 1))],
        axis=0).T
    b2r = b2.reshape(C, 1)

    zacc = jnp.zeros((RPT, H), f32)
    zdeg = jnp.zeros((RPT,), f32)
    ones = jnp.ones((CHUNK,), f32)

    yap, xrp = _mm1(x, wrep, b1x)
    accp, degp = _agg_with_deg(ei3, yap.reshape(NPAD, H), zacc, zdeg, ones)
    accp_pk = accp.reshape(NC, _PK, 128)
    degp_pk = degp.reshape(NC, _PK, 128)
    hp = _fuse1(accp_pk, degp_pk, xrp)
    accp2, = _agg_no_deg(ei3, hp.reshape(NPAD, H), zacc)
    return _fuse2(accp2.reshape(NC, _PK, 128), degp_pk, hp, w2x, b2r).T


# speedup vs baseline: 1.0046x; 1.0046x over previous
import functools

import jax
import jax.numpy as jnp
from jax import lax
from jax.experimental import pallas as pl
from jax.experimental.pallas import tpu as pltpu
from jax.experimental.pallas import tpu_sc as plsc

N = 10000
E = 320000
F_IN = 128
H = 16
C = 40

NPAD = 10240
NC = 2
NS = 16
NW = NC * NS
CHUNK = 128
NCH_TOTAL = E // CHUNK
NCH_BASE = NCH_TOTAL // NW
NCH_EXTRA = NCH_TOTAL % NW
RPT = NPAD // NS
DEPTH = 32
GA = 16
SD = 16

RPACK = 128 // H
RB = 1024
PKR = RB // RPACK
_GRID = NPAD // RB
_PK = NPAD // RPACK


def _agg_body(with_deg, *refs):
    if with_deg:
        (ei_hbm, tbl_hbm, zacc_hbm, zdeg_hbm, ones_hbm,
         acc_out, deg_out,
         srcv, dstv, rows, onesv, degv, degrep,
         acc_sh, deg_sh, sem_g, sem_s, sem_d) = refs
    else:
        (ei_hbm, tbl_hbm, zacc_hbm,
         acc_out,
         srcv, dstv, rows, acc_sh, sem_g, sem_s) = refs

    c = lax.axis_index("c")
    s = lax.axis_index("s")
    w = c * NS + s
    row0 = NCH_BASE * w + jnp.minimum(w, NCH_EXTRA)
    nch = NCH_BASE + jnp.where(w < NCH_EXTRA, 1, 0)

    zb = s * RPT
    init_cps = [
        pltpu.make_async_copy(zacc_hbm, acc_sh.at[pl.ds(zb, RPT)], sem_g),
        pltpu.make_async_copy(ei_hbm.at[0, pl.ds(row0, NCH_BASE)],
                              srcv.at[pl.ds(0, NCH_BASE)], sem_g),
        pltpu.make_async_copy(ei_hbm.at[1, pl.ds(row0, NCH_BASE)],
                              dstv.at[pl.ds(0, NCH_BASE)], sem_g),
    ]
    if with_deg:
        init_cps += [
            pltpu.make_async_copy(zdeg_hbm, deg_sh.at[pl.ds(zb, RPT)], sem_g),
            pltpu.make_async_copy(ones_hbm, onesv, sem_g),
        ]
    for cp in init_cps:
        cp.start()

    @pl.when(w < NCH_EXTRA)
    def _():
        pltpu.sync_copy(ei_hbm.at[0, row0 + NCH_BASE], srcv.at[NCH_BASE])
        pltpu.sync_copy(ei_hbm.at[1, row0 + NCH_BASE], dstv.at[NCH_BASE])

    for cp in init_cps:
        cp.wait()

    plsc.subcore_barrier()

    def g_slot(i):
        return rows.at[pl.ds(lax.rem(i, DEPTH) * CHUNK, CHUNK)]

    for k in range(GA):
        @pl.when(k < nch)
        def _(k=k):
            pltpu.make_async_copy(tbl_hbm.at[srcv.at[k]],
                                  rows.at[pl.ds(k * CHUNK, CHUNK)],
                                  sem_g).start()

    def chunk_step(i):
        @pl.when(i >= SD)
        def _():
            pltpu.make_async_copy(g_slot(i - SD),
                                  acc_sh.at[dstv.at[i - SD]], sem_s).wait()
            if with_deg:
                pltpu.make_async_copy(onesv, deg_sh.at[dstv.at[i - SD]],
                                      sem_d).wait()

        @pl.when(i + GA < nch)
        def _():
            pltpu.make_async_copy(tbl_hbm.at[srcv.at[i + GA]],
                                  g_slot(i + GA), sem_g).start()

        pltpu.make_async_copy(tbl_hbm.at[srcv.at[i]], g_slot(i), sem_g).wait()
        pltpu.async_copy(g_slot(i), acc_sh.at[dstv.at[i]], sem_s, add=True)
        if with_deg:
            pltpu.async_copy(onesv, deg_sh.at[dstv.at[i]], sem_d, add=True)

    def step(i, carry):
        chunk_step(i)
        return carry

    lax.fori_loop(0, nch, step, 0)

    def tail(i, carry):
        @pl.when(i >= 0)
        def _():
            pltpu.make_async_copy(g_slot(i), acc_sh.at[dstv.at[i]],
                                  sem_s).wait()
            if with_deg:
                pltpu.make_async_copy(onesv, deg_sh.at[dstv.at[i]],
                                      sem_d).wait()
        return carry

    lax.fori_loop(jnp.maximum(nch - SD, 0), nch, tail, 0)

    plsc.subcore_barrier()
    pltpu.sync_copy(acc_sh.at[pl.ds(zb, RPT)], acc_out.at[c, pl.ds(zb, RPT)])
    if with_deg:
        pltpu.sync_copy(deg_sh.at[pl.ds(zb, RPT)], degv)

        def rep(i, carry):
            v = degv[pl.ds(i * H, H)]
            for k in range(H):
                degrep[i * H + k, :] = jnp.full((H,), v[k], jnp.float32)
            return carry

        lax.fori_loop(0, RPT // H, rep, 0)
        pltpu.sync_copy(degrep, deg_out.at[c, pl.ds(zb, RPT)])


def _make_agg(with_deg):
    mesh = plsc.VectorSubcoreMesh(
        core_axis_name="c", subcore_axis_name="s",
        num_cores=NC, num_subcores=NS)
    out_type = [jax.ShapeDtypeStruct((NC, NPAD, H), jnp.float32)]
    scratch = [
        pltpu.VMEM((NCH_BASE + 1, CHUNK), jnp.int32),
        pltpu.VMEM((NCH_BASE + 1, CHUNK), jnp.int32),
        pltpu.VMEM((DEPTH * CHUNK, H), jnp.float32),
    ]
    if with_deg:
        out_type.append(jax.ShapeDtypeStruct((NC, NPAD, H), jnp.float32))
        scratch += [
            pltpu.VMEM((CHUNK,), jnp.float32),
            pltpu.VMEM((RPT,), jnp.float32),
            pltpu.VMEM((RPT, H), jnp.float32),
        ]
    scratch.append(pltpu.VMEM_SHARED((NPAD, H), jnp.float32))
    if with_deg:
        scratch.append(pltpu.VMEM_SHARED((NPAD,), jnp.float32))
    scratch += [pltpu.SemaphoreType.DMA, pltpu.SemaphoreType.DMA]
    if with_deg:
        scratch.append(pltpu.SemaphoreType.DMA)
    return pl.kernel(
        functools.partial(_agg_body, with_deg),
        out_type=out_type,
        mesh=mesh,
        scratch_types=scratch,
        compiler_params=pltpu.CompilerParams(use_tc_tiling_on_sc=False),
    )


def _mm1_body(x_ref, w_ref, b_ref, ya_ref, xr_ref):
    z = jnp.dot(x_ref[...], w_ref[...], preferred_element_type=jnp.float32)
    nl = lax.broadcasted_iota(jnp.int32, (RB, 128), 0)
    li = lax.broadcasted_iota(jnp.int32, (RB, 128), 1)
    msk = jnp.where((li >> 4) == (nl & 7), 1.0, 0.0)
    msk2 = jnp.concatenate([msk, msk], axis=1)
    qi = lax.broadcasted_iota(jnp.int32, (PKR, RB), 0)
    ni = lax.broadcasted_iota(jnp.int32, (PKR, RB), 1)
    a8t = jnp.where((ni >> 3) == qi, 1.0, 0.0)
    yz = jnp.dot(a8t, z * msk2, preferred_element_type=jnp.float32)
    ya_ref[...] = yz[:, :128]
    xr_ref[...] = yz[:, 128:] + b_ref[...]


def _fuse1_body(acc_ref, deg_ref, xr_ref, h_ref):
    a = acc_ref[...]
    d = deg_ref[...]
    rinv = 1.0 / jnp.maximum(d[0] + d[1], 1.0)
    h_ref[...] = jnp.maximum((a[0] + a[1]) * rinv + xr_ref[...], 0.0)


def _fuse2_body(acc_ref, deg_ref, h_ref, w_ref, b_ref, out_ref):
    a = acc_ref[...]
    d = deg_ref[...]
    rinv = 1.0 / jnp.maximum(d[0] + d[1], 1.0)
    mean2 = (a[0] + a[1]) * rinv
    m2t = mean2.T
    ht = h_ref[...].T
    qi = lax.broadcasted_iota(jnp.int32, (PKR, RB), 0)
    ni = lax.broadcasted_iota(jnp.int32, (PKR, RB), 1)
    a8x = jnp.where(qi == (ni >> 3), 1.0, 0.0)
    li = lax.broadcasted_iota(jnp.int32, (128, RB), 0)
    nl = lax.broadcasted_iota(jnp.int32, (128, RB), 1)
    mskx = jnp.where((li >> 4) == (nl & 7), 1.0, 0.0)
    m2x = jnp.dot(m2t, a8x, preferred_element_type=jnp.float32) * mskx
    hx = jnp.dot(ht, a8x, preferred_element_type=jnp.float32) * mskx
    hw = jnp.concatenate([m2x, hx], axis=0)
    o = jnp.dot(w_ref[...], hw, preferred_element_type=jnp.float32) + b_ref[...]
    m = jnp.max(o, axis=0, keepdims=True)
    e = jnp.exp(o - m)
    lse = jnp.log(jnp.sum(e, axis=0, keepdims=True))
    out_ref[...] = (o - m) - lse


def _mm1(x, wrep, b1x):
    return pl.pallas_call(
        _mm1_body,
        grid=(_GRID,),
        in_specs=[
            pl.BlockSpec((RB, F_IN), lambda i: (i, 0)),
            pl.BlockSpec((F_IN, 256), lambda i: (0, 0)),
            pl.BlockSpec((1, 128), lambda i: (0, 0)),
        ],
        out_specs=[
            pl.BlockSpec((PKR, 128), lambda i: (i, 0)),
            pl.BlockSpec((PKR, 128), lambda i: (i, 0)),
        ],
        out_shape=[
            jax.ShapeDtypeStruct((_PK, 128), jnp.float32),
            jax.ShapeDtypeStruct((_PK, 128), jnp.float32),
        ],
    )(x, wrep, b1x)


def _fuse1(accp, degp, xrp):
    return pl.pallas_call(
        _fuse1_body,
        grid=(_GRID,),
        in_specs=[
            pl.BlockSpec((NC, PKR, 128), lambda i: (0, i, 0)),
            pl.BlockSpec((NC, PKR, 128), lambda i: (0, i, 0)),
            pl.BlockSpec((PKR, 128), lambda i: (i, 0)),
        ],
        out_specs=pl.BlockSpec((PKR, 128), lambda i: (i, 0)),
        out_shape=jax.ShapeDtypeStruct((_PK, 128), jnp.float32),
    )(accp, degp, xrp)


def _fuse2(accp2, degp, hp, w2x, b2r):
    return pl.pallas_call(
        _fuse2_body,
        grid=(_GRID,),
        in_specs=[
            pl.BlockSpec((NC, PKR, 128), lambda i: (0, i, 0)),
            pl.BlockSpec((NC, PKR, 128), lambda i: (0, i, 0)),
            pl.BlockSpec((PKR, 128), lambda i: (i, 0)),
            pl.BlockSpec((C, 256), lambda i: (0, 0)),
            pl.BlockSpec((C, 1), lambda i: (0, 0)),
        ],
        out_specs=pl.BlockSpec((C, RB), lambda i: (0, i)),
        out_shape=jax.ShapeDtypeStruct((C, N), jnp.float32),
    )(accp2, degp, hp, w2x, b2r)


_agg_with_deg = _make_agg(True)
_agg_no_deg = _make_agg(False)


@jax.jit
def kernel(x, edge_index, W1_l, W1_r, b1, W2_l, W2_r, b2):
    f32 = jnp.float32
    ei3 = edge_index.reshape(2, NCH_TOTAL, CHUNK)

    wrep = jnp.concatenate(
        [jnp.tile(W1_l, (1, RPACK)), jnp.tile(W1_r, (1, RPACK))],
        axis=1)
    b1x = jnp.tile(b1, RPACK).reshape(1, 128)
    w2x = jnp.concatenate(
        [jnp.tile(W2_l, (RPACK, 1)), jnp.tile(W2_r, (RPACK, 1))],
        axis=0).T
    b2r = b2.reshape(C, 1)

    zacc = jnp.zeros((RPT, H), f32)
    zdeg = jnp.zeros((RPT,), f32)
    ones = jnp.ones((CHUNK,), f32)

    yap, xrp = _mm1(x, wrep, b1x)
    accp, degp = _agg_with_deg(ei3, yap.reshape(NPAD, H), zacc, zdeg, ones)
    accp_pk = accp.reshape(NC, _PK, 128)
    degp_pk = degp.reshape(NC, _PK, 128)
    hp = _fuse1(accp_pk, degp_pk, xrp)
    accp2, = _agg_no_deg(ei3, hp.reshape(NPAD, H), zacc)
    return _fuse2(accp2.reshape(NC, _PK, 128), degp_pk, hp, w2x, b2r).T
